# baseline (speedup 1.0000x reference)
import functools

import jax
import jax.numpy as jnp
from jax import lax
from jax.experimental import pallas as pl
from jax.experimental.pallas import tpu as pltpu


def _ceil_to(x, m):
    return ((x + m - 1) // m) * m


def _loss_kernel(e_hbm, m_hbm, a_hbm, out_ref, e_ref, m_ref, buf0, buf1,
                 sem_a, sem_em, *, d_p, n_rel, ch):
    n_p = e_ref.shape[0]
    n_chunks = n_rel // ch
    bufs = (buf0, buf1)

    cp_e = pltpu.make_async_copy(e_hbm, e_ref, sem_em.at[0])
    cp_m = pltpu.make_async_copy(m_hbm, m_ref, sem_em.at[1])
    cp_e.start()
    cp_m.start()

    def start(i):
        pltpu.make_async_copy(a_hbm.at[pl.ds(i * ch, ch)], bufs[i % 2],
                              sem_a.at[i % 2]).start()

    def wait(i):
        pltpu.make_async_copy(a_hbm.at[pl.ds(i * ch, ch)], bufs[i % 2],
                              sem_a.at[i % 2]).wait()

    start(0)
    start(1)

    cp_e.wait()
    e = e_ref[...]
    sq = e * e
    nrm2 = jnp.dot(sq, jnp.ones((d_p, 128), jnp.float32),
                   preferred_element_type=jnp.float32)
    inv = lax.rsqrt(jnp.maximum(nrm2, 1e-24))
    e_nbf = (e * inv).astype(jnp.bfloat16)
    e_ext = jnp.concatenate(
        [e_nbf, jnp.ones((n_p, 8), jnp.bfloat16)], axis=1)
    et = e_ext.T
    ent = et[0:d_p, :]
    g = jnp.dot(ent, e_nbf, preferred_element_type=jnp.float32)
    ones_fold = jnp.ones((8, d_p), jnp.float32)
    cp_m.wait()

    sa_acc = jnp.zeros((1, n_p), jnp.float32)
    bm_acc = jnp.zeros((8, n_p), jnp.float32)
    t_acc = jnp.zeros((d_p, d_p), jnp.float32)
    for i in range(n_chunks):
        if i + 2 < n_chunks:
            start(i + 2)
        wait(i)
        for kk in range(ch):
            a = bufs[i % 2][kk].astype(jnp.bfloat16)
            c = jnp.dot(et, a, preferred_element_type=jnp.float32)
            sa_acc = sa_acc + c[d_p:d_p + 1, :]
            mk = m_ref[i * ch + kk]
            z = jnp.dot(mk.astype(jnp.bfloat16), ent,
                        preferred_element_type=jnp.float32)
            w = c[0:d_p, :] * z
            bm_acc = bm_acc + jnp.dot(ones_fold, w,
                                      preferred_element_type=jnp.float32)
            y1 = jnp.dot(g, mk, preferred_element_type=jnp.float32)
            y2 = jnp.dot(mk, g, preferred_element_type=jnp.float32)
            t_acc = t_acc + y1 * y2

    val = (jnp.sum(sa_acc) - 2.0 * jnp.sum(bm_acc[0:1, :]) + jnp.sum(t_acc))
    out_ref[...] = val + jnp.zeros((1, 128), jnp.float32)


def kernel(E, M, A):
    n, d = E.shape
    K = M.shape[0]

    n_p = _ceil_to(n, 128)
    d_p = _ceil_to(d, 128)
    ch = 2 if K % 2 == 0 else 1

    E_p = E if E.dtype == jnp.float32 else E.astype(jnp.float32)
    M_p = M if M.dtype == jnp.float32 else M.astype(jnp.float32)
    A_p = A
    if d_p != d:
        E_p = jnp.pad(E_p, ((0, 0), (0, d_p - d)))
        M_p = jnp.pad(M_p, ((0, 0), (0, d_p - d), (0, d_p - d)))
    if n_p != n:
        E_p = jnp.pad(E_p, ((0, n_p - n), (0, 0)))
        A_p = jnp.pad(A_p, ((0, 0), (0, n_p - n), (0, n_p - n)))

    out = pl.pallas_call(
        functools.partial(_loss_kernel, d_p=d_p, n_rel=K, ch=ch),
        out_shape=jax.ShapeDtypeStruct((1, 128), jnp.float32),
        in_specs=[
            pl.BlockSpec(memory_space=pl.ANY),
            pl.BlockSpec(memory_space=pl.ANY),
            pl.BlockSpec(memory_space=pl.ANY),
        ],
        out_specs=pl.BlockSpec(memory_space=pltpu.VMEM),
        scratch_shapes=[
            pltpu.VMEM((n_p, d_p), jnp.float32),
            pltpu.VMEM((K, d_p, d_p), jnp.float32),
            pltpu.VMEM((ch, n_p, n_p), jnp.int8),
            pltpu.VMEM((ch, n_p, n_p), jnp.int8),
            pltpu.SemaphoreType.DMA((2,)),
            pltpu.SemaphoreType.DMA((2,)),
        ],
        compiler_params=pltpu.CompilerParams(
            vmem_limit_bytes=63 * 2 ** 20,
        ),
    )(E_p, M_p, A_p)

    return out[0, 0]


# speedup vs baseline: 1.0514x; 1.0514x over previous
import functools

import jax
import jax.numpy as jnp
from jax import lax
from jax.experimental import pallas as pl
from jax.experimental.pallas import tpu as pltpu


def _ceil_to(x, m):
    return ((x + m - 1) // m) * m


def _loss_kernel(e_hbm, m_hbm, a_hbm, out_ref, e_ref, m_ref, buf0, buf1, buf2,
                 sem_a, sem_em, *, d_p, n_rel, ch):
    n_p = e_ref.shape[0]
    n_chunks = n_rel // ch
    bufs = (buf0, buf1, buf2)

    cp_e = pltpu.make_async_copy(e_hbm, e_ref, sem_em.at[0])
    cp_m = pltpu.make_async_copy(m_hbm, m_ref, sem_em.at[1])
    cp_e.start()
    cp_m.start()

    def start(i):
        pltpu.make_async_copy(a_hbm.at[pl.ds(i * ch, ch)], bufs[i % 3],
                              sem_a.at[i % 3]).start()

    def wait(i):
        pltpu.make_async_copy(a_hbm.at[pl.ds(i * ch, ch)], bufs[i % 3],
                              sem_a.at[i % 3]).wait()

    start(0)
    if n_chunks > 1:
        start(1)

    cp_e.wait()
    e = e_ref[...]
    sq = e * e
    nrm2 = jnp.dot(sq, jnp.ones((d_p, 128), jnp.float32),
                   preferred_element_type=jnp.float32)
    inv = lax.rsqrt(jnp.maximum(nrm2, 1e-24))
    e_nbf = (e * inv).astype(jnp.bfloat16)
    e_ext = jnp.concatenate(
        [e_nbf, jnp.ones((n_p, 8), jnp.bfloat16)], axis=1)
    et = e_ext.T
    ent = et[0:d_p, :]
    g = jnp.dot(ent, e_nbf, preferred_element_type=jnp.float32)
    cp_m.wait()

    val = jnp.float32(0.0)
    for i in range(n_chunks):
        if i + 2 < n_chunks:
            start(i + 2)
        wait(i)
        for kk in range(ch):
            a = bufs[i % 3][kk].astype(jnp.bfloat16)
            c = jnp.dot(et, a, preferred_element_type=jnp.float32)
            sum_a = jnp.sum(c[d_p:d_p + 1, :])
            mk = m_ref[i * ch + kk]
            z = jnp.dot(mk.astype(jnp.bfloat16), ent,
                        preferred_element_type=jnp.float32)
            dot_bm = jnp.sum(c[0:d_p, :] * z)
            y1 = jnp.dot(g, mk, preferred_element_type=jnp.float32)
            y2 = jnp.dot(mk, g, preferred_element_type=jnp.float32)
            t3 = jnp.sum(y1 * y2)
            val = val + sum_a - 2.0 * dot_bm + t3

    out_ref[...] = val + jnp.zeros((1, 128), jnp.float32)


def kernel(E, M, A):
    n, d = E.shape
    K = M.shape[0]

    n_p = _ceil_to(n, 128)
    d_p = _ceil_to(d, 128)
    ch = 2 if K % 2 == 0 else 1

    E_p = E if E.dtype == jnp.float32 else E.astype(jnp.float32)
    M_p = M if M.dtype == jnp.float32 else M.astype(jnp.float32)
    A_p = A
    if d_p != d:
        E_p = jnp.pad(E_p, ((0, 0), (0, d_p - d)))
        M_p = jnp.pad(M_p, ((0, 0), (0, d_p - d), (0, d_p - d)))
    if n_p != n:
        E_p = jnp.pad(E_p, ((0, n_p - n), (0, 0)))
        A_p = jnp.pad(A_p, ((0, 0), (0, n_p - n), (0, n_p - n)))

    out = pl.pallas_call(
        functools.partial(_loss_kernel, d_p=d_p, n_rel=K, ch=ch),
        out_shape=jax.ShapeDtypeStruct((1, 128), jnp.float32),
        in_specs=[
            pl.BlockSpec(memory_space=pl.ANY),
            pl.BlockSpec(memory_space=pl.ANY),
            pl.BlockSpec(memory_space=pl.ANY),
        ],
        out_specs=pl.BlockSpec(memory_space=pltpu.VMEM),
        scratch_shapes=[
            pltpu.VMEM((n_p, d_p), jnp.float32),
            pltpu.VMEM((K, d_p, d_p), jnp.float32),
            pltpu.VMEM((ch, n_p, n_p), jnp.int8),
            pltpu.VMEM((ch, n_p, n_p), jnp.int8),
            pltpu.VMEM((ch, n_p, n_p), jnp.int8),
            pltpu.SemaphoreType.DMA((3,)),
            pltpu.SemaphoreType.DMA((2,)),
        ],
        compiler_params=pltpu.CompilerParams(
            vmem_limit_bytes=63 * 2 ** 20,
        ),
    )(E_p, M_p, A_p)

    return out[0, 0]
